# baseline (speedup 1.0000x reference)
import functools
import jax
import jax.numpy as jnp
from jax import lax
from jax.experimental import pallas as pl
from jax.experimental.pallas import tpu as pltpu
from jax.experimental.pallas import tpu_sc as plsc

B, S, D = 4, 4096, 4096
ROWS = B * S
L = 16
DJ = D // L
NC, NS = 2, 16
NW = NC * NS
RC = 8
NBUF = 3


def _lane_perm(v, idx):
    dnums = lax.GatherDimensionNumbers(
        offset_dims=(), collapsed_slice_dims=(0,), start_index_map=(0,))
    return lax.gather(v, idx[:, None], dnums, (1,),
                      mode=lax.GatherScatterMode.PROMISE_IN_BOUNDS)


def _sc_compute_chunk(buf, w_v, bias_v):
    def jbody(j, accs):
        c0 = pl.multiple_of(j * L, L)
        wv = w_v[pl.ds(c0, L)]
        new = []
        for r in range(RC):
            x = buf[r, pl.ds(c0, L)]
            buf[r, pl.ds(c0, L)] = x + wv
            new.append(accs[r] + x * wv)
        return tuple(new)

    init = tuple(bias_v[...] for _ in range(RC))
    accs = lax.fori_loop(0, DJ, jbody, init)
    lane = lax.iota(jnp.int32, L)
    lat_vec = jnp.zeros((L,), jnp.float32)
    for r in range(RC):
        t = accs[r]
        for k in (8, 4, 2, 1):
            t = t + _lane_perm(t, lane ^ k)
        lat_vec = jnp.where(lane == r, jnp.maximum(t, 0.0), lat_vec)
    return lat_vec


def _make_sc_kernel(rows):
    rpw = rows // NW
    nchunk = rpw // RC
    mesh = plsc.VectorSubcoreMesh(core_axis_name="c", subcore_axis_name="s")

    @functools.partial(
        pl.kernel,
        out_type=[
            jax.ShapeDtypeStruct((rows, D), jnp.float32),
            jax.ShapeDtypeStruct((rows // RC, L), jnp.float32),
        ],
        mesh=mesh,
        scratch_types=[
            pltpu.VMEM((D,), jnp.float32),
            pltpu.VMEM((L,), jnp.float32),
            pltpu.VMEM((NBUF, RC, D), jnp.float32),
            pltpu.VMEM((nchunk, L), jnp.float32),
            pltpu.SemaphoreType.DMA,
            pltpu.SemaphoreType.DMA,
            pltpu.SemaphoreType.DMA,
            pltpu.SemaphoreType.DMA,
            pltpu.SemaphoreType.DMA,
            pltpu.SemaphoreType.DMA,
        ],
        compiler_params=pltpu.CompilerParams(use_tc_tiling_on_sc=True),
    )
    def sc_kernel(base_hbm, w_hbm, bias_hbm, out_hbm, lat_hbm,
                  w_v, bias_v, bufs, lat_v,
                  sem_in0, sem_in1, sem_in2, sem_out0, sem_out1, sem_out2):
        sem_in = (sem_in0, sem_in1, sem_in2)
        sem_out = (sem_out0, sem_out1, sem_out2)
        wid = lax.axis_index("s") * NC + lax.axis_index("c")
        row0 = wid * rpw
        pltpu.sync_copy(w_hbm, w_v)
        pltpu.sync_copy(bias_hbm, bias_v)

        def in_cp(cc, b):
            return pltpu.make_async_copy(
                base_hbm.at[pl.ds(row0 + cc * RC, RC)], bufs.at[b], sem_in[b])

        def out_cp(cc, b):
            return pltpu.make_async_copy(
                bufs.at[b], out_hbm.at[pl.ds(row0 + cc * RC, RC)], sem_out[b])

        def process(cc, b, first):
            in_cp(cc, b).wait()
            lat_vec = _sc_compute_chunk(bufs.at[b], w_v, bias_v)
            out_cp(cc, b).start()
            nxt = (b + 2) % NBUF
            if not first:
                out_cp(cc, nxt).wait()

            @pl.when(cc + 2 < nchunk)
            def _():
                in_cp(cc + 2, nxt).start()

            lat_v[cc] = lat_vec

        npeel = NBUF + (nchunk - NBUF) % NBUF
        for b in range(2):
            in_cp(b, b).start()
        for cc in range(npeel):
            process(cc, cc % NBUF, cc == 0)

        @pl.loop(npeel, nchunk, step=NBUF)
        def _(cc):
            for j in range(NBUF):
                process(cc + j, (npeel + j) % NBUF, False)

        out_cp(nchunk - 1, (nchunk - 1) % NBUF).wait()
        pltpu.sync_copy(lat_v, lat_hbm.at[pl.ds(wid * nchunk, nchunk)])

    return sc_kernel


ROWS_SC = 6144
TR = 512


def _tc_body(w_ref, bias_ref, x_ref, out_ref, lat_ref):
    x = x_ref[...]
    w = w_ref[...]
    out_ref[...] = x + w
    acc = jnp.sum(x * w, axis=1) + bias_ref[0]
    lat_ref[0, 0, :] = jnp.maximum(acc, 0.0)


def _tc_call(x2, proj_weight, proj_bias):
    rows = x2.shape[0]
    n_tiles = rows // TR
    out2, lat3 = pl.pallas_call(
        _tc_body,
        grid=(n_tiles,),
        in_specs=[
            pl.BlockSpec((1, D), lambda i: (0, 0)),
            pl.BlockSpec(memory_space=pltpu.SMEM),
            pl.BlockSpec((TR, D), lambda i: (i, 0)),
        ],
        out_specs=[
            pl.BlockSpec((TR, D), lambda i: (i, 0)),
            pl.BlockSpec((1, 1, TR), lambda i: (i, 0, 0)),
        ],
        out_shape=[
            jax.ShapeDtypeStruct((rows, D), jnp.float32),
            jax.ShapeDtypeStruct((n_tiles, 1, TR), jnp.float32),
        ],
    )(proj_weight, proj_bias, x2)
    return out2, lat3.reshape(rows)


def kernel(base, proj_weight, proj_bias):
    x2 = base.reshape(ROWS, D)
    w1 = proj_weight.reshape(D)
    bias16 = jnp.zeros((L,), jnp.float32).at[0].set(proj_bias[0])
    sc_out, sc_lat2 = _make_sc_kernel(ROWS_SC)(x2[:ROWS_SC], w1, bias16)
    tc_out, tc_lat = _tc_call(x2[ROWS_SC:], proj_weight, proj_bias)
    out2 = jnp.concatenate([sc_out, tc_out], axis=0)
    lat = jnp.concatenate([sc_lat2[:, :RC].reshape(ROWS_SC), tc_lat])
    return out2.reshape(B, S, D), lat.reshape(B, S)


# speedup vs baseline: 3.0935x; 3.0935x over previous
import jax
import jax.numpy as jnp
from jax.experimental import pallas as pl
from jax.experimental.pallas import tpu as pltpu

B, S, D = 4, 4096, 4096
ROWS = B * S
TR = 256


def _body(w_ref, bias_ref, x_ref, out_ref, lat_ref):
    x = x_ref[...]
    w = w_ref[...]
    out_ref[...] = x + w
    acc = jnp.sum(x * w, axis=1) + bias_ref[0]
    lat_ref[0, 0, :] = jnp.maximum(acc, 0.0)


def kernel(base, proj_weight, proj_bias):
    n_tiles = ROWS // TR
    x2 = base.reshape(ROWS, D)
    out2, lat3 = pl.pallas_call(
        _body,
        grid=(n_tiles,),
        in_specs=[
            pl.BlockSpec((1, D), lambda i: (0, 0)),
            pl.BlockSpec(memory_space=pltpu.SMEM),
            pl.BlockSpec((TR, D), lambda i: (i, 0)),
        ],
        out_specs=[
            pl.BlockSpec((TR, D), lambda i: (i, 0)),
            pl.BlockSpec((1, 1, TR), lambda i: (i, 0, 0)),
        ],
        out_shape=[
            jax.ShapeDtypeStruct((ROWS, D), base.dtype),
            jax.ShapeDtypeStruct((n_tiles, 1, TR), jnp.float32),
        ],
    )(proj_weight, proj_bias, x2)
    return out2.reshape(B, S, D), lat3.reshape(B, S)
